# baseline (speedup 1.0000x reference)
import functools
import jax
import jax.numpy as jnp
from jax import lax
from jax.experimental import pallas as pl
from jax.experimental.pallas import tpu as pltpu
from jax.experimental.pallas import tpu_sc as plsc

NC, NS = 2, 16
KW = 125


def _sc_mesh():
    return plsc.VectorSubcoreMesh(
        core_axis_name="c", subcore_axis_name="s",
        num_cores=NC, num_subcores=NS)


def _sc_degree(dst3, zeros_hbm, n, d):
    rpt = dst3.shape[1]
    seg = n // NS

    @functools.partial(
        pl.kernel,
        out_type=jax.ShapeDtypeStruct((NC, NS, seg, d), jnp.float32),
        mesh=_sc_mesh(),
        scratch_types=[
            pltpu.VMEM((rpt // 2, KW), jnp.int32),
            pltpu.VMEM((KW, d), jnp.float32),
            pltpu.VMEM_SHARED((n, d), jnp.float32),
        ],
    )
    def deg_kernel(dst_hbm, zeros_h, out_hbm, idx_v, ones_v, acc):
        cid = lax.axis_index("c")
        sid = lax.axis_index("s")
        wid = cid * NS + sid
        half = rpt // 2
        pltpu.sync_copy(zeros_h, acc.at[pl.ds(sid * seg, seg)])

        @pl.loop(0, KW)
        def _(r):
            for c8 in range(d // 16):
                ones_v[r, pl.ds(c8 * 16, 16)] = jnp.full((16,), 1.0,
                                                         jnp.float32)

        plsc.subcore_barrier()

        for h in range(2):
            pltpu.sync_copy(dst_hbm.at[wid, pl.ds(h * half, half)], idx_v)

            @pl.loop(0, half)
            def _(j):
                pltpu.sync_copy(ones_v, acc.at[idx_v.at[j]], add=True)

        plsc.subcore_barrier()
        pltpu.sync_copy(acc.at[pl.ds(sid * seg, seg)],
                        out_hbm.at[cid, sid])

    return deg_kernel(dst3, zeros_hbm).reshape(NC, n, d)


def _sc_scatter(hs, src3, dst3, zeros_hbm, n, d):
    rpt = src3.shape[1]
    seg = n // NS

    @functools.partial(
        pl.kernel,
        out_type=jax.ShapeDtypeStruct((NC, NS, seg, d), jnp.float32),
        mesh=_sc_mesh(),
        scratch_types=[
            pltpu.VMEM((rpt // 2, KW), jnp.int32),
            pltpu.VMEM((rpt // 2, KW), jnp.int32),
            pltpu.VMEM((KW, d), jnp.float32),
            pltpu.VMEM((KW, d), jnp.float32),
            pltpu.VMEM_SHARED((n, d), jnp.float32),
            pltpu.SemaphoreType.DMA,
            pltpu.SemaphoreType.DMA,
        ],
    )
    def scat_kernel(hs_hbm, src_hbm, dst_hbm, z_hbm, out_hbm,
                    src_v, dst_v, rows_a, rows_b, acc, gs_a, gs_b):
        cid = lax.axis_index("c")
        sid = lax.axis_index("s")
        wid = cid * NS + sid
        half = rpt // 2
        pltpu.sync_copy(z_hbm, acc.at[pl.ds(sid * seg, seg)])
        plsc.subcore_barrier()

        for h in range(2):
            pltpu.sync_copy(src_hbm.at[wid, pl.ds(h * half, half)], src_v)
            pltpu.sync_copy(dst_hbm.at[wid, pl.ds(h * half, half)], dst_v)
            @pl.loop(0, half)
            def _(j):
                pltpu.async_copy(hs_hbm.at[src_v.at[j]], rows_a, gs_a).wait()
                pltpu.sync_copy(rows_a, acc.at[dst_v.at[j]], add=True)
        plsc.subcore_barrier()
        pltpu.sync_copy(acc.at[pl.ds(sid * seg, seg)],
                        out_hbm.at[cid, sid])

    return scat_kernel(hs, src3, dst3, zeros_hbm).reshape(NC, n, d)


def _tc_matmul(x, W, n, d, bn):
    g = n // bn

    def body(x_ref, w_ref, h_ref):
        h_ref[...] = jnp.dot(x_ref[...], w_ref[...],
                             preferred_element_type=jnp.float32)

    return pl.pallas_call(
        body,
        grid=(g,),
        in_specs=[
            pl.BlockSpec((bn, d), lambda i: (i, 0)),
            pl.BlockSpec((d, d), lambda i: (0, 0)),
        ],
        out_specs=pl.BlockSpec((bn, d), lambda i: (i, 0)),
        out_shape=jax.ShapeDtypeStruct((n, d), jnp.float32),
    )(x, W)


def _tc_scale(degp, h, n, d, bn):
    g = n // bn

    def body(deg_ref, h_ref, hs_ref, dis_ref):
        deg = deg_ref[0, :, 0:1] + deg_ref[1, :, 0:1] + 1.0
        dis = lax.rsqrt(deg)
        hs_ref[...] = h_ref[...] * dis
        dis_ref[...] = dis

    return pl.pallas_call(
        body,
        grid=(g,),
        in_specs=[
            pl.BlockSpec((NC, bn, d), lambda i: (0, i, 0)),
            pl.BlockSpec((bn, d), lambda i: (i, 0)),
        ],
        out_specs=[
            pl.BlockSpec((bn, d), lambda i: (i, 0)),
            pl.BlockSpec((bn, 1), lambda i: (i, 0)),
        ],
        out_shape=[
            jax.ShapeDtypeStruct((n, d), jnp.float32),
            jax.ShapeDtypeStruct((n, 1), jnp.float32),
        ],
    )(degp, h)


def _tc_mid(part, hs1, dis, b1, xres, W2, n, d, bn):
    g = n // bn

    def body(p_ref, hs_ref, dis_ref, b_ref, xr_ref, w_ref, h1_ref, hs2_ref):
        s = p_ref[0] + p_ref[1] + hs_ref[...]
        h1 = jnp.maximum(s * dis_ref[...] + b_ref[...], 0.0) + xr_ref[...]
        h1_ref[...] = h1
        hs2_ref[...] = jnp.dot(h1, w_ref[...],
                               preferred_element_type=jnp.float32) * dis_ref[...]

    return pl.pallas_call(
        body,
        grid=(g,),
        in_specs=[
            pl.BlockSpec((NC, bn, d), lambda i: (0, i, 0)),
            pl.BlockSpec((bn, d), lambda i: (i, 0)),
            pl.BlockSpec((bn, 1), lambda i: (i, 0)),
            pl.BlockSpec((1, d), lambda i: (0, 0)),
            pl.BlockSpec((bn, d), lambda i: (i, 0)),
            pl.BlockSpec((d, d), lambda i: (0, 0)),
        ],
        out_specs=[
            pl.BlockSpec((bn, d), lambda i: (i, 0)),
            pl.BlockSpec((bn, d), lambda i: (i, 0)),
        ],
        out_shape=[
            jax.ShapeDtypeStruct((n, d), jnp.float32),
            jax.ShapeDtypeStruct((n, d), jnp.float32),
        ],
    )(part, hs1, dis, b1, xres, W2)


def _tc_post(part, hs2, dis, b2, hres, n, d, bn):
    g = n // bn

    def body(p_ref, hs_ref, dis_ref, b_ref, hr_ref, out_ref):
        s = p_ref[0] + p_ref[1] + hs_ref[...]
        out_ref[...] = jnp.maximum(
            s * dis_ref[...] + b_ref[...], 0.0) + hr_ref[...]

    return pl.pallas_call(
        body,
        grid=(g,),
        in_specs=[
            pl.BlockSpec((NC, bn, d), lambda i: (0, i, 0)),
            pl.BlockSpec((bn, d), lambda i: (i, 0)),
            pl.BlockSpec((bn, 1), lambda i: (i, 0)),
            pl.BlockSpec((1, d), lambda i: (0, 0)),
            pl.BlockSpec((bn, d), lambda i: (i, 0)),
        ],
        out_specs=pl.BlockSpec((bn, d), lambda i: (i, 0)),
        out_shape=jax.ShapeDtypeStruct((n, d), jnp.float32),
    )(part, hs2, dis, b2, hres)


def kernel(x, edge_index, edge_type, W1, b1, W2, b2):
    n, d = x.shape
    e = edge_index.shape[1]
    assert e % (NC * NS * KW) == 0 and n % NS == 0 and n % 8 == 0

    rpt = e // (NC * NS * KW)
    src3 = edge_index[0].astype(jnp.int32).reshape(NC * NS, rpt, KW)
    dst3 = edge_index[1].astype(jnp.int32).reshape(NC * NS, rpt, KW)

    seg = n // NS
    zeros_row = jnp.zeros((seg, d), jnp.float32)
    b1r = b1.reshape(1, d)
    b2r = b2.reshape(1, d)

    bn = 1000 if n % 1000 == 0 else seg

    h1raw = _tc_matmul(x, W1, n, d, bn)
    degp = _sc_degree(dst3, zeros_row, n, d)
    hs1, dis = _tc_scale(degp, h1raw, n, d, bn)
    part1 = _sc_scatter(hs1, src3, dst3, zeros_row, n, d)
    h1, hs2 = _tc_mid(part1, hs1, dis, b1r, x, W2, n, d, bn)
    part2 = _sc_scatter(hs2, src3, dst3, zeros_row, n, d)
    h2 = _tc_post(part2, hs2, dis, b2r, h1, n, d, bn)
    return h2


# speedup vs baseline: 1.0018x; 1.0018x over previous
import functools
import jax
import jax.numpy as jnp
from jax import lax
from jax.experimental import pallas as pl
from jax.experimental.pallas import tpu as pltpu
from jax.experimental.pallas import tpu_sc as plsc

NC, NS = 2, 16
KW = 125


def _sc_mesh():
    return plsc.VectorSubcoreMesh(
        core_axis_name="c", subcore_axis_name="s",
        num_cores=NC, num_subcores=NS)


def _sc_degree(dst3, zeros_hbm, n, d):
    rpt = dst3.shape[1]
    seg = n // NS

    @functools.partial(
        pl.kernel,
        out_type=jax.ShapeDtypeStruct((NC, NS, seg, d), jnp.float32),
        mesh=_sc_mesh(),
        scratch_types=[
            pltpu.VMEM((rpt // 2, KW), jnp.int32),
            pltpu.VMEM((KW, d), jnp.float32),
            pltpu.VMEM_SHARED((n, d), jnp.float32),
        ],
    )
    def deg_kernel(dst_hbm, zeros_h, out_hbm, idx_v, ones_v, acc):
        cid = lax.axis_index("c")
        sid = lax.axis_index("s")
        wid = cid * NS + sid
        half = rpt // 2
        pltpu.sync_copy(zeros_h, acc.at[pl.ds(sid * seg, seg)])

        @pl.loop(0, KW)
        def _(r):
            for c8 in range(d // 16):
                ones_v[r, pl.ds(c8 * 16, 16)] = jnp.full((16,), 1.0,
                                                         jnp.float32)

        plsc.subcore_barrier()

        for h in range(2):
            pltpu.sync_copy(dst_hbm.at[wid, pl.ds(h * half, half)], idx_v)

            @pl.loop(0, half)
            def _(j):
                pltpu.sync_copy(ones_v, acc.at[idx_v.at[j]], add=True)

        plsc.subcore_barrier()
        pltpu.sync_copy(acc.at[pl.ds(sid * seg, seg)],
                        out_hbm.at[cid, sid])

    return deg_kernel(dst3, zeros_hbm).reshape(NC, n, d)


def _sc_scatter(hs, src3, dst3, zeros_hbm, n, d):
    rpt = src3.shape[1]
    seg = n // NS

    @functools.partial(
        pl.kernel,
        out_type=jax.ShapeDtypeStruct((NC, NS, seg, d), jnp.float32),
        mesh=_sc_mesh(),
        scratch_types=[
            pltpu.VMEM((rpt // 2, KW), jnp.int32),
            pltpu.VMEM((rpt // 2, KW), jnp.int32),
            pltpu.VMEM((KW, d), jnp.float32),
            pltpu.VMEM((KW, d), jnp.float32),
            pltpu.VMEM_SHARED((n, d), jnp.float32),
            pltpu.SemaphoreType.DMA,
            pltpu.SemaphoreType.DMA,
        ],
    )
    def scat_kernel(hs_hbm, src_hbm, dst_hbm, z_hbm, out_hbm,
                    src_v, dst_v, rows_a, rows_b, acc, gs_a, gs_b):
        cid = lax.axis_index("c")
        sid = lax.axis_index("s")
        wid = cid * NS + sid
        half = rpt // 2
        pltpu.sync_copy(z_hbm, acc.at[pl.ds(sid * seg, seg)])
        plsc.subcore_barrier()

        for h in range(2):
            pltpu.sync_copy(src_hbm.at[wid, pl.ds(h * half, half)], src_v)
            pltpu.sync_copy(dst_hbm.at[wid, pl.ds(h * half, half)], dst_v)
            @pl.loop(0, half)
            def _(j):
                pltpu.async_copy(hs_hbm.at[src_v.at[j]], rows_a, gs_a).wait()
                pltpu.sync_copy(rows_a, acc.at[dst_v.at[j]], add=True)
        plsc.subcore_barrier()
        pltpu.sync_copy(acc.at[pl.ds(sid * seg, seg)],
                        out_hbm.at[cid, sid])

    return scat_kernel(hs, src3, dst3, zeros_hbm).reshape(NC, n, d)


def _tc_pre(degp, x, W1, n, d, bn):
    g = n // bn

    def body(deg_ref, x_ref, w_ref, hs_ref, dis_ref):
        deg = deg_ref[0, :, 0:1] + deg_ref[1, :, 0:1] + 1.0
        dis = lax.rsqrt(deg)
        h = jnp.dot(x_ref[...], w_ref[...],
                    preferred_element_type=jnp.float32)
        hs_ref[...] = h * dis
        dis_ref[...] = dis

    return pl.pallas_call(
        body,
        grid=(g,),
        in_specs=[
            pl.BlockSpec((NC, bn, d), lambda i: (0, i, 0)),
            pl.BlockSpec((bn, d), lambda i: (i, 0)),
            pl.BlockSpec((d, d), lambda i: (0, 0)),
        ],
        out_specs=[
            pl.BlockSpec((bn, d), lambda i: (i, 0)),
            pl.BlockSpec((bn, 1), lambda i: (i, 0)),
        ],
        out_shape=[
            jax.ShapeDtypeStruct((n, d), jnp.float32),
            jax.ShapeDtypeStruct((n, 1), jnp.float32),
        ],
    )(degp, x, W1)


def _tc_mid(part, hs1, dis, b1, xres, W2, n, d, bn):
    g = n // bn

    def body(p_ref, hs_ref, dis_ref, b_ref, xr_ref, w_ref, h1_ref, hs2_ref):
        s = p_ref[0] + p_ref[1] + hs_ref[...]
        h1 = jnp.maximum(s * dis_ref[...] + b_ref[...], 0.0) + xr_ref[...]
        h1_ref[...] = h1
        hs2_ref[...] = jnp.dot(h1, w_ref[...],
                               preferred_element_type=jnp.float32) * dis_ref[...]

    return pl.pallas_call(
        body,
        grid=(g,),
        in_specs=[
            pl.BlockSpec((NC, bn, d), lambda i: (0, i, 0)),
            pl.BlockSpec((bn, d), lambda i: (i, 0)),
            pl.BlockSpec((bn, 1), lambda i: (i, 0)),
            pl.BlockSpec((1, d), lambda i: (0, 0)),
            pl.BlockSpec((bn, d), lambda i: (i, 0)),
            pl.BlockSpec((d, d), lambda i: (0, 0)),
        ],
        out_specs=[
            pl.BlockSpec((bn, d), lambda i: (i, 0)),
            pl.BlockSpec((bn, d), lambda i: (i, 0)),
        ],
        out_shape=[
            jax.ShapeDtypeStruct((n, d), jnp.float32),
            jax.ShapeDtypeStruct((n, d), jnp.float32),
        ],
    )(part, hs1, dis, b1, xres, W2)


def _tc_post(part, hs2, dis, b2, hres, n, d, bn):
    g = n // bn

    def body(p_ref, hs_ref, dis_ref, b_ref, hr_ref, out_ref):
        s = p_ref[0] + p_ref[1] + hs_ref[...]
        out_ref[...] = jnp.maximum(
            s * dis_ref[...] + b_ref[...], 0.0) + hr_ref[...]

    return pl.pallas_call(
        body,
        grid=(g,),
        in_specs=[
            pl.BlockSpec((NC, bn, d), lambda i: (0, i, 0)),
            pl.BlockSpec((bn, d), lambda i: (i, 0)),
            pl.BlockSpec((bn, 1), lambda i: (i, 0)),
            pl.BlockSpec((1, d), lambda i: (0, 0)),
            pl.BlockSpec((bn, d), lambda i: (i, 0)),
        ],
        out_specs=pl.BlockSpec((bn, d), lambda i: (i, 0)),
        out_shape=jax.ShapeDtypeStruct((n, d), jnp.float32),
    )(part, hs2, dis, b2, hres)


def kernel(x, edge_index, edge_type, W1, b1, W2, b2):
    n, d = x.shape
    e = edge_index.shape[1]
    assert e % (NC * NS * KW) == 0 and n % NS == 0 and n % 8 == 0

    rpt = e // (NC * NS * KW)
    src3 = edge_index[0].astype(jnp.int32).reshape(NC * NS, rpt, KW)
    dst3 = edge_index[1].astype(jnp.int32).reshape(NC * NS, rpt, KW)

    seg = n // NS
    zeros_row = jnp.zeros((seg, d), jnp.float32)
    b1r = b1.reshape(1, d)
    b2r = b2.reshape(1, d)

    bn = 1000 if n % 1000 == 0 else seg

    degp = _sc_degree(dst3, zeros_row, n, d)
    hs1, dis = _tc_pre(degp, x, W1, n, d, bn)
    part1 = _sc_scatter(hs1, src3, dst3, zeros_row, n, d)
    h1, hs2 = _tc_mid(part1, hs1, dis, b1r, x, W2, n, d, bn)
    part2 = _sc_scatter(hs2, src3, dst3, zeros_row, n, d)
    h2 = _tc_post(part2, hs2, dis, b2r, h1, n, d, bn)
    return h2


# speedup vs baseline: 1.1133x; 1.1113x over previous
import functools
import jax
import jax.numpy as jnp
from jax import lax
from jax.experimental import pallas as pl
from jax.experimental.pallas import tpu as pltpu
from jax.experimental.pallas import tpu_sc as plsc

NC, NS = 2, 16
KW = 125


def _sc_mesh():
    return plsc.VectorSubcoreMesh(
        core_axis_name="c", subcore_axis_name="s",
        num_cores=NC, num_subcores=NS)


def _sc_degree(dst3, zeros_hbm, n, d):
    rpt = dst3.shape[1]
    seg = n // NS

    @functools.partial(
        pl.kernel,
        out_type=jax.ShapeDtypeStruct((NC, NS, seg, d), jnp.float32),
        mesh=_sc_mesh(),
        scratch_types=[
            pltpu.VMEM((rpt // 2, KW), jnp.int32),
            pltpu.VMEM((KW, d), jnp.float32),
            pltpu.VMEM_SHARED((n, d), jnp.float32),
        ],
    )
    def deg_kernel(dst_hbm, zeros_h, out_hbm, idx_v, ones_v, acc):
        cid = lax.axis_index("c")
        sid = lax.axis_index("s")
        wid = cid * NS + sid
        half = rpt // 2
        pltpu.sync_copy(zeros_h, acc.at[pl.ds(sid * seg, seg)])

        @pl.loop(0, KW)
        def _(r):
            for c8 in range(d // 16):
                ones_v[r, pl.ds(c8 * 16, 16)] = jnp.full((16,), 1.0,
                                                         jnp.float32)

        plsc.subcore_barrier()

        for h in range(2):
            pltpu.sync_copy(dst_hbm.at[wid, pl.ds(h * half, half)], idx_v)

            @pl.loop(0, half)
            def _(j):
                pltpu.sync_copy(ones_v, acc.at[idx_v.at[j]], add=True)

        plsc.subcore_barrier()
        pltpu.sync_copy(acc.at[pl.ds(sid * seg, seg)],
                        out_hbm.at[cid, sid])

    return deg_kernel(dst3, zeros_hbm).reshape(NC, n, d)


def _sc_scatter(hs, src3, dst3, zeros_hbm, n, d):
    rpt = src3.shape[1]
    seg = n // NS

    @functools.partial(
        pl.kernel,
        out_type=jax.ShapeDtypeStruct((NC, NS, seg, d), jnp.float32),
        mesh=_sc_mesh(),
        scratch_types=[
            pltpu.VMEM((rpt // 2, KW), jnp.int32),
            pltpu.VMEM((rpt // 2, KW), jnp.int32),
            pltpu.VMEM((KW, d), jnp.float32),
            pltpu.VMEM((KW, d), jnp.float32),
            pltpu.VMEM_SHARED((n, d), jnp.float32),
            pltpu.SemaphoreType.DMA,
            pltpu.SemaphoreType.DMA,
        ],
    )
    def scat_kernel(hs_hbm, src_hbm, dst_hbm, z_hbm, out_hbm,
                    src_v, dst_v, rows_a, rows_b, acc, gs_a, gs_b):
        cid = lax.axis_index("c")
        sid = lax.axis_index("s")
        wid = cid * NS + sid
        half = rpt // 2
        pltpu.sync_copy(z_hbm, acc.at[pl.ds(sid * seg, seg)])
        plsc.subcore_barrier()

        for h in range(2):
            pltpu.sync_copy(src_hbm.at[wid, pl.ds(h * half, half)], src_v)
            pltpu.sync_copy(dst_hbm.at[wid, pl.ds(h * half, half)], dst_v)
            if half % 2 == 0:
                @pl.loop(0, half // 2)
                def _(i):
                    j0 = i * 2
                    ca = pltpu.async_copy(hs_hbm.at[src_v.at[j0]],
                                          rows_a, gs_a)
                    cb = pltpu.async_copy(hs_hbm.at[src_v.at[j0 + 1]],
                                          rows_b, gs_b)
                    ca.wait()
                    pltpu.sync_copy(rows_a, acc.at[dst_v.at[j0]], add=True)
                    cb.wait()
                    pltpu.sync_copy(rows_b, acc.at[dst_v.at[j0 + 1]],
                                    add=True)
            else:
                @pl.loop(0, half)
                def _(j):
                    pltpu.async_copy(hs_hbm.at[src_v.at[j]],
                                     rows_a, gs_a).wait()
                    pltpu.sync_copy(rows_a, acc.at[dst_v.at[j]], add=True)
        plsc.subcore_barrier()
        pltpu.sync_copy(acc.at[pl.ds(sid * seg, seg)],
                        out_hbm.at[cid, sid])

    return scat_kernel(hs, src3, dst3, zeros_hbm).reshape(NC, n, d)


def _tc_pre(degp, x, W1, n, d, bn):
    g = n // bn

    def body(deg_ref, x_ref, w_ref, hs_ref, dis_ref):
        deg = deg_ref[0, :, 0:1] + deg_ref[1, :, 0:1] + 1.0
        dis = lax.rsqrt(deg)
        h = jnp.dot(x_ref[...], w_ref[...],
                    preferred_element_type=jnp.float32)
        hs_ref[...] = h * dis
        dis_ref[...] = dis

    return pl.pallas_call(
        body,
        grid=(g,),
        in_specs=[
            pl.BlockSpec((NC, bn, d), lambda i: (0, i, 0)),
            pl.BlockSpec((bn, d), lambda i: (i, 0)),
            pl.BlockSpec((d, d), lambda i: (0, 0)),
        ],
        out_specs=[
            pl.BlockSpec((bn, d), lambda i: (i, 0)),
            pl.BlockSpec((bn, 1), lambda i: (i, 0)),
        ],
        out_shape=[
            jax.ShapeDtypeStruct((n, d), jnp.float32),
            jax.ShapeDtypeStruct((n, 1), jnp.float32),
        ],
    )(degp, x, W1)


def _tc_mid(part, hs1, dis, b1, xres, W2, n, d, bn):
    g = n // bn

    def body(p_ref, hs_ref, dis_ref, b_ref, xr_ref, w_ref, h1_ref, hs2_ref):
        s = p_ref[0] + p_ref[1] + hs_ref[...]
        h1 = jnp.maximum(s * dis_ref[...] + b_ref[...], 0.0) + xr_ref[...]
        h1_ref[...] = h1
        hs2_ref[...] = jnp.dot(h1, w_ref[...],
                               preferred_element_type=jnp.float32) * dis_ref[...]

    return pl.pallas_call(
        body,
        grid=(g,),
        in_specs=[
            pl.BlockSpec((NC, bn, d), lambda i: (0, i, 0)),
            pl.BlockSpec((bn, d), lambda i: (i, 0)),
            pl.BlockSpec((bn, 1), lambda i: (i, 0)),
            pl.BlockSpec((1, d), lambda i: (0, 0)),
            pl.BlockSpec((bn, d), lambda i: (i, 0)),
            pl.BlockSpec((d, d), lambda i: (0, 0)),
        ],
        out_specs=[
            pl.BlockSpec((bn, d), lambda i: (i, 0)),
            pl.BlockSpec((bn, d), lambda i: (i, 0)),
        ],
        out_shape=[
            jax.ShapeDtypeStruct((n, d), jnp.float32),
            jax.ShapeDtypeStruct((n, d), jnp.float32),
        ],
    )(part, hs1, dis, b1, xres, W2)


def _tc_post(part, hs2, dis, b2, hres, n, d, bn):
    g = n // bn

    def body(p_ref, hs_ref, dis_ref, b_ref, hr_ref, out_ref):
        s = p_ref[0] + p_ref[1] + hs_ref[...]
        out_ref[...] = jnp.maximum(
            s * dis_ref[...] + b_ref[...], 0.0) + hr_ref[...]

    return pl.pallas_call(
        body,
        grid=(g,),
        in_specs=[
            pl.BlockSpec((NC, bn, d), lambda i: (0, i, 0)),
            pl.BlockSpec((bn, d), lambda i: (i, 0)),
            pl.BlockSpec((bn, 1), lambda i: (i, 0)),
            pl.BlockSpec((1, d), lambda i: (0, 0)),
            pl.BlockSpec((bn, d), lambda i: (i, 0)),
        ],
        out_specs=pl.BlockSpec((bn, d), lambda i: (i, 0)),
        out_shape=jax.ShapeDtypeStruct((n, d), jnp.float32),
    )(part, hs2, dis, b2, hres)


def kernel(x, edge_index, edge_type, W1, b1, W2, b2):
    n, d = x.shape
    e = edge_index.shape[1]
    assert e % (NC * NS * KW) == 0 and n % NS == 0 and n % 8 == 0

    rpt = e // (NC * NS * KW)
    src3 = edge_index[0].astype(jnp.int32).reshape(NC * NS, rpt, KW)
    dst3 = edge_index[1].astype(jnp.int32).reshape(NC * NS, rpt, KW)

    seg = n // NS
    zeros_row = jnp.zeros((seg, d), jnp.float32)
    b1r = b1.reshape(1, d)
    b2r = b2.reshape(1, d)

    bn = 1000 if n % 1000 == 0 else seg

    degp = _sc_degree(dst3, zeros_row, n, d)
    hs1, dis = _tc_pre(degp, x, W1, n, d, bn)
    part1 = _sc_scatter(hs1, src3, dst3, zeros_row, n, d)
    h1, hs2 = _tc_mid(part1, hs1, dis, b1r, x, W2, n, d, bn)
    part2 = _sc_scatter(hs2, src3, dst3, zeros_row, n, d)
    h2 = _tc_post(part2, hs2, dis, b2r, h1, n, d, bn)
    return h2
